# baseline (speedup 1.0000x reference)
import jax
import jax.numpy as jnp
from jax.experimental import pallas as pl

B, F, C = 4096, 100, 100
BB = 32


def _onehot_body(idx_ref, out_ref):
    idx = idx_ref[...]
    classes = jax.lax.broadcasted_iota(jnp.int32, (BB, F, C), 2)
    out_ref[...] = (idx[:, :, None] == classes).astype(jnp.int32)


def kernel(tensor):
    idx2 = tensor.reshape(B, F)
    return pl.pallas_call(
        _onehot_body,
        grid=(B // BB,),
        in_specs=[pl.BlockSpec((BB, F), lambda i: (i, 0))],
        out_specs=pl.BlockSpec((BB, F, C), lambda i: (i, 0, 0)),
        out_shape=jax.ShapeDtypeStruct((B, F, C), jnp.int32),
    )(idx2)


# speedup vs baseline: 1.1125x; 1.1125x over previous
import jax
import jax.numpy as jnp
from jax.experimental import pallas as pl

B, F, C = 4096, 100, 100
BB = 128
LANES = 128


def _onehot_body(idx_ref, out_ref):
    ones = jnp.ones((1, LANES), jnp.float32)
    iota = jax.lax.broadcasted_iota(jnp.int32, (F, C), 1).astype(jnp.float32)
    for b in range(BB):
        x = idx_ref[b : b + 1, :]
        splat = jax.lax.dot_general(
            x, ones, (((0,), (0,)), ((), ())),
            preferred_element_type=jnp.float32,
        )
        out_ref[b] = (splat[:, :C] == iota).astype(jnp.int32)


def kernel(tensor):
    idxf = tensor.reshape(B, F).astype(jnp.float32)
    return pl.pallas_call(
        _onehot_body,
        grid=(B // BB,),
        in_specs=[pl.BlockSpec((BB, F), lambda i: (i, 0))],
        out_specs=pl.BlockSpec((BB, F, C), lambda i: (i, 0, 0)),
        out_shape=jax.ShapeDtypeStruct((B, F, C), jnp.int32),
    )(idxf)


# speedup vs baseline: 1.1209x; 1.0076x over previous
import jax
import jax.numpy as jnp
from jax.experimental import pallas as pl

B, F, C = 4096, 100, 100
BB = 512
LANES = 128


def _onehot_body(idx_ref, out_ref):
    ones = jnp.ones((1, LANES), jnp.float32)
    iota = jax.lax.broadcasted_iota(jnp.int32, (F, C), 1).astype(jnp.float32)
    for b in range(BB):
        x = idx_ref[b : b + 1, :]
        splat = jax.lax.dot_general(
            x, ones, (((0,), (0,)), ((), ())),
            preferred_element_type=jnp.float32,
        )
        out_ref[b] = (splat[:, :C] == iota).astype(jnp.int32)


def kernel(tensor):
    idxf = tensor.reshape(B, F).astype(jnp.float32)
    return pl.pallas_call(
        _onehot_body,
        grid=(B // BB,),
        in_specs=[pl.BlockSpec((BB, F), lambda i: (i, 0))],
        out_specs=pl.BlockSpec((BB, F, C), lambda i: (i, 0, 0)),
        out_shape=jax.ShapeDtypeStruct((B, F, C), jnp.int32),
    )(idxf)
